# speedup vs baseline: 1.8739x; 1.8739x over previous
import jax
import jax.numpy as jnp
from jax.experimental import pallas as pl
from jax.experimental.pallas import tpu as pltpu

B = 1024
D = 1024
K = 8192
LR = 0.001
EPS = 1e-8
TK = 1024


def _assign_body(dp_ref, c_ref, maxv_ref, idx_ref):
    kt = pl.program_id(0)
    dp = dp_ref[...]
    c = c_ref[...]
    num = jax.lax.dot_general(
        dp, c, (((1,), (1,)), ((), ())), preferred_element_type=jnp.float32
    )
    xn = jnp.sqrt(jnp.sum(dp * dp, axis=1, keepdims=True))
    cn = jnp.sqrt(jnp.sum(c * c, axis=1, keepdims=True))
    denom = jnp.maximum(xn * cn.reshape(1, TK), EPS)
    scores = num / denom
    tmax = jnp.max(scores, axis=1, keepdims=True)
    col = jax.lax.broadcasted_iota(jnp.int32, (B, TK), 1)
    targ = jnp.min(
        jnp.where(scores == tmax, col, K), axis=1, keepdims=True
    ) + kt * TK

    @pl.when(kt == 0)
    def _():
        maxv_ref[...] = tmax
        idx_ref[...] = targ

    @pl.when(kt > 0)
    def _():
        better = tmax > maxv_ref[...]
        maxv_ref[...] = jnp.where(better, tmax, maxv_ref[...])
        idx_ref[...] = jnp.where(better, targ, idx_ref[...])


def _update_body(dp_ref, idx_ref, c_ref, out_ref):
    kt = pl.program_id(0)
    dp = dp_ref[...]
    idx = idx_ref[...]
    c = c_ref[...]
    local = idx - kt * TK
    col = jax.lax.broadcasted_iota(jnp.int32, (B, TK), 1)
    onehot = (col == local).astype(jnp.float32)
    sums = jax.lax.dot_general(
        onehot, dp, (((0,), (0,)), ((), ())), preferred_element_type=jnp.float32
    )
    ones = jnp.ones((B, 1), dtype=jnp.float32)
    cnt = jax.lax.dot_general(
        onehot, ones, (((0,), (0,)), ((), ())), preferred_element_type=jnp.float32
    )
    mean = sums / jnp.maximum(cnt, 1.0)
    out_ref[...] = jnp.where(cnt > 0.0, c + LR * (mean - c), c)


def kernel(datapoints, batch_cos_sim, centroid):
    del batch_cos_sim
    dp = datapoints
    _, idx = pl.pallas_call(
        _assign_body,
        grid=(K // TK,),
        in_specs=[
            pl.BlockSpec((B, D), lambda k: (0, 0)),
            pl.BlockSpec((TK, D), lambda k: (k, 0)),
        ],
        out_specs=[
            pl.BlockSpec((B, 1), lambda k: (0, 0)),
            pl.BlockSpec((B, 1), lambda k: (0, 0)),
        ],
        out_shape=[
            jax.ShapeDtypeStruct((B, 1), jnp.float32),
            jax.ShapeDtypeStruct((B, 1), jnp.int32),
        ],
    )(dp, centroid)
    out = pl.pallas_call(
        _update_body,
        grid=(K // TK,),
        in_specs=[
            pl.BlockSpec((B, D), lambda k: (0, 0)),
            pl.BlockSpec((B, 1), lambda k: (0, 0)),
            pl.BlockSpec((TK, D), lambda k: (k, 0)),
        ],
        out_specs=pl.BlockSpec((TK, D), lambda k: (k, 0)),
        out_shape=jax.ShapeDtypeStruct((K, D), jnp.float32),
    )(dp, idx, centroid)
    return out
